# baseline (speedup 1.0000x reference)
import functools

import jax
import jax.numpy as jnp
from jax import lax
from jax.experimental import pallas as pl
from jax.experimental.pallas import tpu as pltpu
from jax.experimental.pallas import tpu_sc as plsc

_NC, _NS, _L = 2, 16, 16
_NW = _NC * _NS

_GDN = lax.GatherDimensionNumbers(
    offset_dims=(), collapsed_slice_dims=(0,), start_index_map=(0,)
)


def _lane_bcast(vec16, lane):
    idx = jnp.full((_L, 1), lane, jnp.int32)
    return lax.gather(
        vec16,
        idx,
        _GDN,
        (1,),
        mode=lax.GatherScatterMode.PROMISE_IN_BOUNDS,
    )


def _make_sc_combine(T, E, D):
    TPW = T // _NW
    C = 4

    mesh = plsc.VectorSubcoreMesh(core_axis_name="c", subcore_axis_name="s")

    @functools.partial(
        pl.kernel,
        mesh=mesh,
        out_type=jax.ShapeDtypeStruct((T, D), jnp.float32),
        scratch_types=[
            pltpu.VMEM((TPW * E,), jnp.float32),
            pltpu.VMEM((C, E, D), jnp.float32),
            pltpu.VMEM((C, D), jnp.float32),
        ],
    )
    def sc_combine(x_hbm, w_hbm, out_hbm, w_v, x_v, o_v):
        wid = lax.axis_index("s") * _NC + lax.axis_index("c")
        base = wid * TPW
        pltpu.sync_copy(w_hbm.at[pl.ds(base * E, TPW * E)], w_v)

        def chunk_body(k, carry):
            t0 = base + k * C
            pltpu.sync_copy(x_hbm.at[pl.ds(t0, C)], x_v)
            for cp in range(0, C, 2):
                wv16 = w_v[pl.ds((k * C + cp) * E, _L)]
                wvecs = [_lane_bcast(wv16, e) for e in range(2 * E)]

                def dloop(db, c2):
                    s = db * _L
                    for ct in (cp, cp + 1):
                        woff = (ct - cp) * E
                        acc = x_v[ct, 0, pl.ds(s, _L)] * wvecs[woff]
                        for e in range(1, E):
                            acc = acc + x_v[ct, e, pl.ds(s, _L)] * wvecs[woff + e]
                        o_v[ct, pl.ds(s, _L)] = acc
                    return c2

                lax.fori_loop(0, D // _L, dloop, 0)
            pltpu.sync_copy(o_v, out_hbm.at[pl.ds(t0, C)])
            return carry

        lax.fori_loop(0, TPW // C, chunk_body, 0)

    return sc_combine


def kernel(expert_outputs_TED, weights_TE):
    T, E, D = expert_outputs_TED.shape
    return _make_sc_combine(T, E, D)(
        expert_outputs_TED, weights_TE.reshape(T * E)
    )


# speedup vs baseline: 2.8805x; 2.8805x over previous
import functools

import jax
import jax.numpy as jnp
from jax import lax
from jax.experimental import pallas as pl
from jax.experimental.pallas import tpu as pltpu
from jax.experimental.pallas import tpu_sc as plsc

_NC, _NS, _L = 2, 16, 16
_NW = _NC * _NS

_GDN = lax.GatherDimensionNumbers(
    offset_dims=(), collapsed_slice_dims=(0,), start_index_map=(0,)
)


def _lane_bcast(vec16, lane):
    idx = jnp.full((_L, 1), lane, jnp.int32)
    return lax.gather(
        vec16,
        idx,
        _GDN,
        (1,),
        mode=lax.GatherScatterMode.PROMISE_IN_BOUNDS,
    )


def _make_sc_combine(T, E, D):
    TPW = T // _NW
    C = 2
    NCH = TPW // C
    NPAIR = NCH // 2
    UNROLL = 4

    mesh = plsc.VectorSubcoreMesh(core_axis_name="c", subcore_axis_name="s")

    @functools.partial(
        pl.kernel,
        mesh=mesh,
        out_type=jax.ShapeDtypeStruct((T, D), jnp.float32),
        scratch_types=[
            pltpu.VMEM((TPW * E,), jnp.float32),
            pltpu.VMEM((C, E, D), jnp.float32),
            pltpu.VMEM((C, E, D), jnp.float32),
            pltpu.VMEM((C, D), jnp.float32),
            pltpu.VMEM((C, D), jnp.float32),
            pltpu.SemaphoreType.DMA,
            pltpu.SemaphoreType.DMA,
            pltpu.SemaphoreType.DMA,
            pltpu.SemaphoreType.DMA,
        ],
    )
    def sc_combine(
        x_hbm, w_hbm, out_hbm, w_v, x_v0, x_v1, o_v0, o_v1, si0, si1, so0, so1
    ):
        wid = lax.axis_index("s") * _NC + lax.axis_index("c")
        base = wid * TPW
        pltpu.sync_copy(w_hbm.at[pl.ds(base * E, TPW * E)], w_v)

        def compute_chunk(xb, ob, c):
            for cp in range(0, C, 2):
                wv16 = w_v[pl.ds((c * C + cp) * E, _L)]
                wvecs = [_lane_bcast(wv16, e) for e in range(2 * E)]

                @plsc.parallel_loop(0, D // _L, unroll=UNROLL)
                def dloop(db):
                    s = db * _L
                    for ct in (cp, cp + 1):
                        woff = (ct - cp) * E
                        acc = xb[ct, 0, pl.ds(s, _L)] * wvecs[woff]
                        for e in range(1, E):
                            acc = acc + xb[ct, e, pl.ds(s, _L)] * wvecs[woff + e]
                        ob[ct, pl.ds(s, _L)] = acc

        def in_slice(c):
            return x_hbm.at[pl.ds(base + c * C, C)]

        def out_slice(c):
            return out_hbm.at[pl.ds(base + c * C, C)]

        pltpu.async_copy(in_slice(0), x_v0, si0)

        def pair_body(j, carry):
            c0 = 2 * j
            c1 = c0 + 1
            pltpu.async_copy(in_slice(c1), x_v1, si1)
            pltpu.make_async_copy(in_slice(c0), x_v0, si0).wait()

            @pl.when(j > 0)
            def _():
                pltpu.make_async_copy(o_v0, out_slice(c0), so0).wait()

            compute_chunk(x_v0, o_v0, c0)

            @pl.when(j + 1 < NPAIR)
            def _():
                pltpu.async_copy(in_slice(c0 + 2), x_v0, si0)

            pltpu.async_copy(o_v0, out_slice(c0), so0)

            pltpu.make_async_copy(in_slice(c1), x_v1, si1).wait()

            @pl.when(j > 0)
            def _():
                pltpu.make_async_copy(o_v1, out_slice(c1), so1).wait()

            compute_chunk(x_v1, o_v1, c1)
            pltpu.async_copy(o_v1, out_slice(c1), so1)
            return carry

        lax.fori_loop(0, NPAIR, pair_body, 0)
        pltpu.make_async_copy(o_v0, out_slice(NCH - 2), so0).wait()
        pltpu.make_async_copy(o_v1, out_slice(NCH - 1), so1).wait()

    return sc_combine


def kernel(expert_outputs_TED, weights_TE):
    T, E, D = expert_outputs_TED.shape
    return _make_sc_combine(T, E, D)(
        expert_outputs_TED, weights_TE.reshape(T * E)
    )
